# baseline (speedup 1.0000x reference)
import functools

import jax
import jax.numpy as jnp
from jax.experimental import pallas as pl
from jax.experimental.pallas import tpu as pltpu

_BR = 80
_RSLOTS = 4
_WSLOTS = 3


def _body(x_ref, w1_ref, b1_ref, w2_ref, b2_ref,
          wmu_ref, bmu_ref, wlv_ref, blv_ref, eps_ref,
          wa1_ref, ba1_ref, wa2_ref, ba2_ref,
          ws1_ref, bs1_ref, ws2_ref, bs2_ref,
          adj_hbm, ahat_hbm, xhat_ref,
          enc_s, s_s, adj_buf, ahat_buf, in_sem, out_sem,
          *, n_rows, h_dim):
    f32 = jnp.float32
    nb = n_rows // _BR
    total_reads = 2 * nb

    def read_copy(r):
        row = (r % nb) * _BR
        slot = r % _RSLOTS
        return pltpu.make_async_copy(
            adj_hbm.at[pl.ds(row, _BR), :], adj_buf.at[slot],
            in_sem.at[slot])

    def start_read(r):
        @pl.when(r < total_reads)
        def _():
            read_copy(r).start()

    def write_copy(k):
        slot = k % _WSLOTS
        return pltpu.make_async_copy(
            ahat_buf.at[slot], ahat_hbm.at[pl.ds(k * _BR, _BR), :],
            out_sem.at[slot])

    enc_s[:, :h_dim] = jnp.dot(x_ref[...], w1_ref[...],
                               preferred_element_type=f32)

    read_copy(0).start()
    read_copy(1).start()
    read_copy(2).start()

    def p2_step(k, carry):
        read_copy(k).wait()
        h = jnp.dot(adj_buf[k % _RSLOTS], enc_s[:, :h_dim],
                    preferred_element_type=f32)
        h = jax.nn.relu(h + b1_ref[...])
        enc_s[pl.ds(k * _BR, _BR), h_dim:] = jnp.dot(
            h, w2_ref[...], preferred_element_type=f32)
        start_read(k + 3)
        return carry

    jax.lax.fori_loop(0, nb, p2_step, 0, unroll=False)

    def p3_step(k, carry):
        r = nb + k
        read_copy(r).wait()
        row = k * _BR
        h = jnp.dot(adj_buf[r % _RSLOTS], enc_s[:, h_dim:],
                    preferred_element_type=f32)
        h = jax.nn.relu(h + b2_ref[...])
        mu = jnp.dot(h, wmu_ref[...], preferred_element_type=f32) + bmu_ref[...]
        lv = jnp.dot(h, wlv_ref[...], preferred_element_type=f32) + blv_ref[...]
        z = mu + eps_ref[pl.ds(row, _BR), :] * jnp.exp(0.5 * lv)
        a = jax.nn.relu(
            jnp.dot(z, wa1_ref[...], preferred_element_type=f32) + ba1_ref[...])
        xhat_ref[pl.ds(row, _BR), :] = jnp.dot(
            a, wa2_ref[...], preferred_element_type=f32) + ba2_ref[...]
        s = jax.nn.relu(
            jnp.dot(z, ws1_ref[...], preferred_element_type=f32) + bs1_ref[...])
        s_s[pl.ds(row, _BR), :] = jnp.dot(
            s, ws2_ref[...], preferred_element_type=f32) + bs2_ref[...]
        start_read(r + 3)
        return carry

    jax.lax.fori_loop(0, nb, p3_step, 0, unroll=False)

    def p4_step(k, carry):
        @pl.when(k >= _WSLOTS)
        def _():
            write_copy(k - _WSLOTS).wait()
        logits = jax.lax.dot_general(
            s_s[pl.ds(k * _BR, _BR), :], s_s[...], (((1,), (1,)), ((), ())),
            preferred_element_type=f32)
        ahat_buf[k % _WSLOTS] = jax.nn.sigmoid(logits)
        write_copy(k).start()
        return carry

    jax.lax.fori_loop(0, nb, p4_step, 0, unroll=False)
    write_copy(nb - 3).wait()
    write_copy(nb - 2).wait()
    write_copy(nb - 1).wait()


def kernel(x, adj, W1, b1, W2, b2, Wmu, bmu, Wlv, blv,
           Wa1, ba1, Wa2, ba2, Ws1, bs1, Ws2, bs2):
    N, F = x.shape
    H = W1.shape[1]
    L = Wmu.shape[1]
    f32 = jnp.float32

    b1r = b1.reshape(1, H); b2r = b2.reshape(1, H)
    bmur = bmu.reshape(1, L); blvr = blv.reshape(1, L)
    ba1r = ba1.reshape(1, L); ba2r = ba2.reshape(1, F)
    bs1r = bs1.reshape(1, L); bs2r = bs2.reshape(1, L)
    eps = jax.random.normal(jax.random.key(42), (N, L), f32)

    def vmem(a):
        return pl.BlockSpec(a.shape, lambda: (0, 0))

    A_hat, x_hat = pl.pallas_call(
        functools.partial(_body, n_rows=N, h_dim=H),
        grid=(),
        in_specs=[
            vmem(x), vmem(W1), vmem(b1r), vmem(W2), vmem(b2r),
            vmem(Wmu), vmem(bmur), vmem(Wlv), vmem(blvr), vmem(eps),
            vmem(Wa1), vmem(ba1r), vmem(Wa2), vmem(ba2r),
            vmem(Ws1), vmem(bs1r), vmem(Ws2), vmem(bs2r),
            pl.BlockSpec(memory_space=pl.ANY),
        ],
        out_specs=[
            pl.BlockSpec(memory_space=pl.ANY),
            pl.BlockSpec((N, F), lambda: (0, 0)),
        ],
        out_shape=[jax.ShapeDtypeStruct((N, N), f32),
                   jax.ShapeDtypeStruct((N, F), f32)],
        scratch_shapes=[
            pltpu.VMEM((N, 2 * H), f32),
            pltpu.VMEM((N, L), f32),
            pltpu.VMEM((_RSLOTS, _BR, N), f32),
            pltpu.VMEM((_WSLOTS, _BR, N), f32),
            pltpu.SemaphoreType.DMA((_RSLOTS,)),
            pltpu.SemaphoreType.DMA((_WSLOTS,)),
        ],
    )(x, W1, b1r, W2, b2r, Wmu, bmur, Wlv, blvr, eps,
      Wa1, ba1r, Wa2, ba2r, Ws1, bs1r, Ws2, bs2r, adj)

    return (A_hat, x_hat)


# speedup vs baseline: 1.1313x; 1.1313x over previous
import functools

import jax
import jax.numpy as jnp
from jax.experimental import pallas as pl
from jax.experimental.pallas import tpu as pltpu

_BR = 200
_RSLOTS = 3
_WSLOTS = 2


def _body(x_ref, w1_ref, b1_ref, w2_ref, b2_ref,
          wmu_ref, bmu_ref, wlv_ref, blv_ref, eps_ref,
          wa1_ref, ba1_ref, wa2_ref, ba2_ref,
          ws1_ref, bs1_ref, ws2_ref, bs2_ref,
          adj_hbm, ahat_hbm, xhat_ref,
          enc_s, adj_buf, ahat_buf, in_sem, out_sem,
          *, n_rows, h_dim, l_dim):
    f32 = jnp.float32
    nb = n_rows // _BR
    total_reads = 2 * nb

    def read_copy(r):
        row = (r % nb) * _BR
        slot = r % _RSLOTS
        return pltpu.make_async_copy(
            adj_hbm.at[pl.ds(row, _BR), :], adj_buf.at[slot],
            in_sem.at[slot])

    def start_read(r):
        @pl.when(r < total_reads)
        def _():
            read_copy(r).start()

    def write_copy(k):
        slot = k % _WSLOTS
        return pltpu.make_async_copy(
            ahat_buf.at[slot], ahat_hbm.at[pl.ds(k * _BR, _BR), :],
            out_sem.at[slot])

    enc_s[:, :h_dim] = jnp.dot(x_ref[...], w1_ref[...],
                               preferred_element_type=f32)

    read_copy(0).start()
    read_copy(1).start()
    read_copy(2).start()

    def p2_step(k, carry):
        read_copy(k).wait()
        h = jnp.dot(adj_buf[k % _RSLOTS], enc_s[:, :h_dim],
                    preferred_element_type=f32)
        h = jax.nn.relu(h + b1_ref[...])
        enc_s[pl.ds(k * _BR, _BR), h_dim:] = jnp.dot(
            h, w2_ref[...], preferred_element_type=f32)
        start_read(k + _RSLOTS)
        return carry

    jax.lax.fori_loop(0, nb, p2_step, 0, unroll=False)

    def p3_step(k, carry):
        r = nb + k
        read_copy(r).wait()
        row = k * _BR
        h = jnp.dot(adj_buf[r % _RSLOTS], enc_s[:, h_dim:],
                    preferred_element_type=f32)
        h = jax.nn.relu(h + b2_ref[...])
        mu = jnp.dot(h, wmu_ref[...], preferred_element_type=f32) + bmu_ref[...]
        lv = jnp.dot(h, wlv_ref[...], preferred_element_type=f32) + blv_ref[...]
        z = mu + eps_ref[pl.ds(row, _BR), :] * jnp.exp(0.5 * lv)
        a = jax.nn.relu(
            jnp.dot(z, wa1_ref[...], preferred_element_type=f32) + ba1_ref[...])
        xhat_ref[pl.ds(row, _BR), :] = jnp.dot(
            a, wa2_ref[...], preferred_element_type=f32) + ba2_ref[...]
        s = jax.nn.relu(
            jnp.dot(z, ws1_ref[...], preferred_element_type=f32) + bs1_ref[...])
        enc_s[pl.ds(row, _BR), :l_dim] = jnp.dot(
            s, ws2_ref[...], preferred_element_type=f32) + bs2_ref[...]
        start_read(r + _RSLOTS)
        return carry

    jax.lax.fori_loop(0, nb, p3_step, 0, unroll=False)

    def p4_step(k, carry):
        @pl.when(k >= _WSLOTS)
        def _():
            write_copy(k - _WSLOTS).wait()
        logits = jax.lax.dot_general(
            enc_s[pl.ds(k * _BR, _BR), :l_dim], enc_s[:, :l_dim],
            (((1,), (1,)), ((), ())), preferred_element_type=f32)
        ahat_buf[k % _WSLOTS] = jax.nn.sigmoid(logits)
        write_copy(k).start()
        return carry

    jax.lax.fori_loop(0, nb, p4_step, 0, unroll=False)
    write_copy(nb - 2).wait()
    write_copy(nb - 1).wait()


def kernel(x, adj, W1, b1, W2, b2, Wmu, bmu, Wlv, blv,
           Wa1, ba1, Wa2, ba2, Ws1, bs1, Ws2, bs2):
    N, F = x.shape
    H = W1.shape[1]
    L = Wmu.shape[1]
    f32 = jnp.float32

    b1r = b1.reshape(1, H); b2r = b2.reshape(1, H)
    bmur = bmu.reshape(1, L); blvr = blv.reshape(1, L)
    ba1r = ba1.reshape(1, L); ba2r = ba2.reshape(1, F)
    bs1r = bs1.reshape(1, L); bs2r = bs2.reshape(1, L)
    eps = jax.random.normal(jax.random.key(42), (N, L), f32)

    def vmem(a):
        return pl.BlockSpec(a.shape, lambda: (0, 0))

    A_hat, x_hat = pl.pallas_call(
        functools.partial(_body, n_rows=N, h_dim=H, l_dim=L),
        grid=(),
        in_specs=[
            vmem(x), vmem(W1), vmem(b1r), vmem(W2), vmem(b2r),
            vmem(Wmu), vmem(bmur), vmem(Wlv), vmem(blvr), vmem(eps),
            vmem(Wa1), vmem(ba1r), vmem(Wa2), vmem(ba2r),
            vmem(Ws1), vmem(bs1r), vmem(Ws2), vmem(bs2r),
            pl.BlockSpec(memory_space=pl.ANY),
        ],
        out_specs=[
            pl.BlockSpec(memory_space=pl.ANY),
            pl.BlockSpec((N, F), lambda: (0, 0)),
        ],
        out_shape=[jax.ShapeDtypeStruct((N, N), f32),
                   jax.ShapeDtypeStruct((N, F), f32)],
        scratch_shapes=[
            pltpu.VMEM((N, 2 * H), f32),
            pltpu.VMEM((_RSLOTS, _BR, N), f32),
            pltpu.VMEM((_WSLOTS, _BR, N), f32),
            pltpu.SemaphoreType.DMA((_RSLOTS,)),
            pltpu.SemaphoreType.DMA((_WSLOTS,)),
        ],
    )(x, W1, b1r, W2, b2r, Wmu, bmur, Wlv, blvr, eps,
      Wa1, ba1r, Wa2, ba2r, Ws1, bs1r, Ws2, bs2r, adj)

    return (A_hat, x_hat)


# speedup vs baseline: 1.1830x; 1.0456x over previous
import functools

import jax
import jax.numpy as jnp
from jax.experimental import pallas as pl
from jax.experimental.pallas import tpu as pltpu

_BR = 200
_RSLOTS = 3
_WSLOTS = 2
_H1 = 96


def _body(x_ref, w1_ref, b1_ref, w2_ref, b2_ref,
          wml_ref, bml_ref, eps_ref,
          was_ref, bas_ref, wa2_ref, ba2_ref, ws2_ref, bs2_ref,
          adj_hbm, ahat_hbm, xhat_hbm,
          enc_s, adj_buf, ahat_buf, xh_buf, in_sem, out_sem, xh_sem,
          *, n_rows, h_dim, l_dim):
    f32 = jnp.float32
    nb = n_rows // _BR
    total_reads = 2 * nb

    def read_copy(r):
        row = (r % nb) * _BR
        slot = r % _RSLOTS
        return pltpu.make_async_copy(
            adj_hbm.at[pl.ds(row, _BR), :], adj_buf.at[slot],
            in_sem.at[slot])

    def start_read(r):
        @pl.when(r < total_reads)
        def _():
            read_copy(r).start()

    def xh_copy(k):
        slot = k % 2
        return pltpu.make_async_copy(
            xh_buf.at[slot], xhat_hbm.at[pl.ds(k * _BR, _BR), :],
            xh_sem.at[slot])

    def write_half(k, half):
        slot = k % _WSLOTS
        off = 0 if half == 0 else _H1
        rows = _H1 if half == 0 else _BR - _H1
        return pltpu.make_async_copy(
            ahat_buf.at[slot, pl.ds(off, rows), :],
            ahat_hbm.at[pl.ds(k * _BR + off, rows), :],
            out_sem.at[slot])

    enc_s[:, :h_dim] = jnp.dot(x_ref[...], w1_ref[...],
                               preferred_element_type=f32)

    read_copy(0).start()
    read_copy(1).start()
    read_copy(2).start()

    def p2_step(k, carry):
        read_copy(k).wait()
        h = jnp.dot(adj_buf[k % _RSLOTS], enc_s[:, :h_dim],
                    preferred_element_type=f32)
        start_read(k + _RSLOTS)
        h = jax.nn.relu(h + b1_ref[...])
        enc_s[pl.ds(k * _BR, _BR), h_dim:] = jnp.dot(
            h, w2_ref[...], preferred_element_type=f32)
        return carry

    jax.lax.fori_loop(0, nb, p2_step, 0, unroll=False)

    def p3_step(k, carry):
        r = nb + k
        read_copy(r).wait()
        row = k * _BR
        h = jnp.dot(adj_buf[r % _RSLOTS], enc_s[:, h_dim:],
                    preferred_element_type=f32)
        start_read(r + _RSLOTS)
        h = jax.nn.relu(h + b2_ref[...])
        ml = jnp.dot(h, wml_ref[...], preferred_element_type=f32) + bml_ref[...]
        z = ml[:, :l_dim] + eps_ref[pl.ds(row, _BR), :] * jnp.exp(
            0.5 * ml[:, l_dim:])
        az = jax.nn.relu(
            jnp.dot(z, was_ref[...], preferred_element_type=f32) + bas_ref[...])
        @pl.when(k >= 2)
        def _():
            xh_copy(k - 2).wait()
        xh_buf[k % 2] = jnp.dot(
            az[:, :l_dim], wa2_ref[...], preferred_element_type=f32) + ba2_ref[...]
        xh_copy(k).start()
        enc_s[pl.ds(row, _BR), :l_dim] = jnp.dot(
            az[:, l_dim:], ws2_ref[...], preferred_element_type=f32) + bs2_ref[...]
        return carry

    jax.lax.fori_loop(0, nb, p3_step, 0, unroll=False)
    xh_copy(nb - 2).wait()
    xh_copy(nb - 1).wait()

    def p4_step(k, carry):
        slot = k % _WSLOTS

        @pl.when(k >= _WSLOTS)
        def _():
            write_half(k - _WSLOTS, 0).wait()
            write_half(k - _WSLOTS, 1).wait()

        s_all = enc_s[:, :l_dim]
        row = k * _BR
        lg0 = jax.lax.dot_general(
            enc_s[pl.ds(row, _H1), :l_dim], s_all,
            (((1,), (1,)), ((), ())), preferred_element_type=f32)
        ahat_buf[slot, pl.ds(0, _H1), :] = jax.nn.sigmoid(lg0)
        write_half(k, 0).start()
        lg1 = jax.lax.dot_general(
            enc_s[pl.ds(row + _H1, _BR - _H1), :l_dim], s_all,
            (((1,), (1,)), ((), ())), preferred_element_type=f32)
        ahat_buf[slot, pl.ds(_H1, _BR - _H1), :] = jax.nn.sigmoid(lg1)
        write_half(k, 1).start()
        return carry

    jax.lax.fori_loop(0, nb, p4_step, 0, unroll=False)
    write_half(nb - 2, 0).wait()
    write_half(nb - 2, 1).wait()
    write_half(nb - 1, 0).wait()
    write_half(nb - 1, 1).wait()


def kernel(x, adj, W1, b1, W2, b2, Wmu, bmu, Wlv, blv,
           Wa1, ba1, Wa2, ba2, Ws1, bs1, Ws2, bs2):
    N, F = x.shape
    H = W1.shape[1]
    L = Wmu.shape[1]
    f32 = jnp.float32

    b1r = b1.reshape(1, H); b2r = b2.reshape(1, H)
    Wml = jnp.concatenate([Wmu, Wlv], axis=1)
    bml = jnp.concatenate([bmu, blv]).reshape(1, 2 * L)
    Was = jnp.concatenate([Wa1, Ws1], axis=1)
    bas = jnp.concatenate([ba1, bs1]).reshape(1, 2 * L)
    ba2r = ba2.reshape(1, F)
    bs2r = bs2.reshape(1, L)
    eps = jax.random.normal(jax.random.key(42), (N, L), f32)

    def vmem(a):
        return pl.BlockSpec(a.shape, lambda: (0, 0))

    A_hat, x_hat = pl.pallas_call(
        functools.partial(_body, n_rows=N, h_dim=H, l_dim=L),
        grid=(),
        in_specs=[
            vmem(x), vmem(W1), vmem(b1r), vmem(W2), vmem(b2r),
            vmem(Wml), vmem(bml), vmem(eps),
            vmem(Was), vmem(bas), vmem(Wa2), vmem(ba2r),
            vmem(Ws2), vmem(bs2r),
            pl.BlockSpec(memory_space=pl.ANY),
        ],
        out_specs=[
            pl.BlockSpec(memory_space=pl.ANY),
            pl.BlockSpec(memory_space=pl.ANY),
        ],
        out_shape=[jax.ShapeDtypeStruct((N, N), f32),
                   jax.ShapeDtypeStruct((N, F), f32)],
        scratch_shapes=[
            pltpu.VMEM((N, 2 * H), f32),
            pltpu.VMEM((_RSLOTS, _BR, N), f32),
            pltpu.VMEM((_WSLOTS, _BR, N), f32),
            pltpu.VMEM((2, _BR, F), f32),
            pltpu.SemaphoreType.DMA((_RSLOTS,)),
            pltpu.SemaphoreType.DMA((_WSLOTS,)),
            pltpu.SemaphoreType.DMA((2,)),
        ],
    )(x, W1, b1r, W2, b2r, Wml, bml, eps,
      Was, bas, Wa2, ba2r, Ws2, bs2r, adj)

    return (A_hat, x_hat)


# speedup vs baseline: 1.2027x; 1.0167x over previous
import functools

import jax
import jax.numpy as jnp
from jax.experimental import pallas as pl
from jax.experimental.pallas import tpu as pltpu

_BR = 200
_RSLOTS = 3
_WSLOTS = 2
_H1 = 96


def _body(x_ref, w1_ref, b1_ref, w2_ref, b2_ref,
          wml_ref, bml_ref, eps_ref,
          was_ref, bas_ref, wa2_ref, ba2_ref, ws2_ref, bs2_ref,
          adj_hbm, ahat_hbm, xhat_hbm,
          enc_s, adj_buf, ahat_buf, xh_buf, in_sem, out_sem, xh_sem,
          *, n_rows, h_dim, l_dim):
    f32 = jnp.float32
    nb = n_rows // _BR
    total_reads = 2 * nb

    def read_copy(r):
        row = (r % nb) * _BR
        slot = r % _RSLOTS
        return pltpu.make_async_copy(
            adj_hbm.at[pl.ds(row, _BR), :], adj_buf.at[slot],
            in_sem.at[slot])

    def start_read(r):
        @pl.when(r < total_reads)
        def _():
            read_copy(r).start()

    def xh_copy(k):
        slot = k % 2
        return pltpu.make_async_copy(
            xh_buf.at[slot], xhat_hbm.at[pl.ds(k * _BR, _BR), :],
            xh_sem.at[slot])

    def write_half(k, half):
        slot = k % _WSLOTS
        off = 0 if half == 0 else _H1
        rows = _H1 if half == 0 else _BR - _H1
        return pltpu.make_async_copy(
            ahat_buf.at[slot, pl.ds(off, rows), :],
            ahat_hbm.at[pl.ds(k * _BR + off, rows), :],
            out_sem.at[slot])

    enc_s[:, :h_dim] = jnp.dot(x_ref[...], w1_ref[...],
                               preferred_element_type=f32)

    read_copy(0).start()
    read_copy(1).start()
    read_copy(2).start()

    def p2_step(k, carry):
        read_copy(k).wait()
        h = jnp.dot(adj_buf[k % _RSLOTS], enc_s[:, :h_dim],
                    preferred_element_type=f32)
        start_read(k + _RSLOTS)
        h = jax.nn.relu(h + b1_ref[...])
        enc_s[pl.ds(k * _BR, _BR), h_dim:] = jnp.dot(
            h, w2_ref[...], preferred_element_type=f32)
        return carry

    jax.lax.fori_loop(0, nb, p2_step, 0, unroll=False)

    def p3_step(k, carry):
        r = nb + k
        read_copy(r).wait()
        row = k * _BR
        h = jnp.dot(adj_buf[r % _RSLOTS], enc_s[:, h_dim:],
                    preferred_element_type=f32)
        start_read(r + _RSLOTS)
        h = jax.nn.relu(h + b2_ref[...])
        ml = jnp.dot(h, wml_ref[...], preferred_element_type=f32) + bml_ref[...]
        z = ml[:, :l_dim] + eps_ref[pl.ds(row, _BR), :] * jnp.exp(
            0.5 * ml[:, l_dim:])
        az = jax.nn.relu(
            jnp.dot(z, was_ref[...], preferred_element_type=f32) + bas_ref[...])
        @pl.when(k >= 2)
        def _():
            xh_copy(k - 2).wait()
        xh_buf[k % 2] = jnp.dot(
            az[:, :l_dim], wa2_ref[...], preferred_element_type=f32) + ba2_ref[...]
        xh_copy(k).start()
        enc_s[pl.ds(row, _BR), :l_dim] = jnp.dot(
            az[:, l_dim:], ws2_ref[...], preferred_element_type=f32) + bs2_ref[...]
        return carry

    jax.lax.fori_loop(0, nb, p3_step, 0, unroll=False)
    xh_copy(nb - 2).wait()
    xh_copy(nb - 1).wait()

    def p4_step(k, carry):
        slot = k % _WSLOTS

        @pl.when(k >= _WSLOTS)
        def _():
            write_half(k - _WSLOTS, 0).wait()
            write_half(k - _WSLOTS, 1).wait()

        s_all = enc_s[:, :l_dim]
        row = k * _BR
        lg0 = jax.lax.dot_general(
            enc_s[pl.ds(row, _H1), :l_dim], s_all,
            (((1,), (1,)), ((), ())), preferred_element_type=f32)
        ahat_buf[slot, pl.ds(0, _H1), :] = 0.5 * jnp.tanh(0.5 * lg0) + 0.5
        write_half(k, 0).start()
        lg1 = jax.lax.dot_general(
            enc_s[pl.ds(row + _H1, _BR - _H1), :l_dim], s_all,
            (((1,), (1,)), ((), ())), preferred_element_type=f32)
        ahat_buf[slot, pl.ds(_H1, _BR - _H1), :] = 0.5 * jnp.tanh(0.5 * lg1) + 0.5
        write_half(k, 1).start()
        return carry

    jax.lax.fori_loop(0, nb, p4_step, 0, unroll=False)
    write_half(nb - 2, 0).wait()
    write_half(nb - 2, 1).wait()
    write_half(nb - 1, 0).wait()
    write_half(nb - 1, 1).wait()


def kernel(x, adj, W1, b1, W2, b2, Wmu, bmu, Wlv, blv,
           Wa1, ba1, Wa2, ba2, Ws1, bs1, Ws2, bs2):
    N, F = x.shape
    H = W1.shape[1]
    L = Wmu.shape[1]
    f32 = jnp.float32

    b1r = b1.reshape(1, H); b2r = b2.reshape(1, H)
    Wml = jnp.concatenate([Wmu, Wlv], axis=1)
    bml = jnp.concatenate([bmu, blv]).reshape(1, 2 * L)
    Was = jnp.concatenate([Wa1, Ws1], axis=1)
    bas = jnp.concatenate([ba1, bs1]).reshape(1, 2 * L)
    ba2r = ba2.reshape(1, F)
    bs2r = bs2.reshape(1, L)
    eps = jax.random.normal(jax.random.key(42), (N, L), f32)

    def vmem(a):
        return pl.BlockSpec(a.shape, lambda: (0, 0))

    A_hat, x_hat = pl.pallas_call(
        functools.partial(_body, n_rows=N, h_dim=H, l_dim=L),
        grid=(),
        in_specs=[
            vmem(x), vmem(W1), vmem(b1r), vmem(W2), vmem(b2r),
            vmem(Wml), vmem(bml), vmem(eps),
            vmem(Was), vmem(bas), vmem(Wa2), vmem(ba2r),
            vmem(Ws2), vmem(bs2r),
            pl.BlockSpec(memory_space=pl.ANY),
        ],
        out_specs=[
            pl.BlockSpec(memory_space=pl.ANY),
            pl.BlockSpec(memory_space=pl.ANY),
        ],
        out_shape=[jax.ShapeDtypeStruct((N, N), f32),
                   jax.ShapeDtypeStruct((N, F), f32)],
        scratch_shapes=[
            pltpu.VMEM((N, 2 * H), f32),
            pltpu.VMEM((_RSLOTS, _BR, N), f32),
            pltpu.VMEM((_WSLOTS, _BR, N), f32),
            pltpu.VMEM((2, _BR, F), f32),
            pltpu.SemaphoreType.DMA((_RSLOTS,)),
            pltpu.SemaphoreType.DMA((_WSLOTS,)),
            pltpu.SemaphoreType.DMA((2,)),
        ],
    )(x, W1, b1r, W2, b2r, Wml, bml, eps,
      Was, bas, Wa2, ba2r, Ws2, bs2r, adj)

    return (A_hat, x_hat)
